# Initial kernel scaffold; baseline (speedup 1.0000x reference)
import jax
import jax.numpy as jnp
from jax.experimental import pallas as pl


def kernel(x, edge_index, W, B):
    raise NotImplementedError("write your pallas kernel here")


# speedup vs baseline: 5.1586x; 5.1586x over previous
import functools

import jax
import jax.numpy as jnp
from jax import lax
from jax.experimental import pallas as pl
from jax.experimental.pallas import tpu as pltpu
from jax.experimental.pallas import tpu_sc as plsc

N_NODES = 10000
N_EDGES = 320000
D = 128
DE = 144

NC = 2
NS = 16
NW = NC * NS
EPW = N_EDGES // NW
CH = 80
NCH = EPW // CH
RPT = N_NODES // NS


def _mm_ext_body(x_ref, w_ref, o_ref):
    y = lax.dot_general(x_ref[...], w_ref[...], (((1,), (1,)), ((), ())),
                        preferred_element_type=jnp.float32)
    o_ref[:, :D] = y
    col = lax.broadcasted_iota(jnp.int32, (x_ref.shape[0], DE - D), 1)
    o_ref[:, D:] = jnp.where(col == 0, 1.0, 0.0).astype(jnp.float32)


def _finish_body(a0_ref, a1_ref, x_ref, b_ref, o_ref):
    a = a0_ref[...] + a1_ref[...]
    agg = a[:, :D]
    deg = jnp.maximum(a[:, D:D + 1], 1.0)
    xb = lax.dot_general(x_ref[...], b_ref[...], (((1,), (1,)), ((), ())),
                         preferred_element_type=jnp.float32)
    o_ref[...] = agg / deg + xb


def _sc_scatter_body(yext, zeros_hbm, src_hbm, dst_hbm, out0, out1,
                     src_v, dst_v, rows_v, agg_sh, sem):
    c = lax.axis_index("c")
    s = lax.axis_index("s")
    wid = s * NC + c

    pltpu.sync_copy(zeros_hbm.at[pl.ds(s * RPT, RPT)],
                    agg_sh.at[pl.ds(s * RPT, RPT)])
    plsc.subcore_barrier()

    ebase = wid * EPW

    def body(i, carry):
        off = ebase + i * CH
        pltpu.sync_copy(src_hbm.at[pl.ds(off, CH)], src_v)
        pltpu.sync_copy(dst_hbm.at[pl.ds(off, CH)], dst_v)
        pltpu.async_copy(yext.at[src_v], rows_v, sem).wait()
        pltpu.sync_copy(rows_v, agg_sh.at[dst_v], add=True)
        return carry

    lax.fori_loop(0, NCH, body, 0)
    plsc.subcore_barrier()

    @pl.when(c == 0)
    def _():
        pltpu.sync_copy(agg_sh.at[pl.ds(s * RPT, RPT)],
                        out0.at[pl.ds(s * RPT, RPT)])

    @pl.when(c == 1)
    def _():
        pltpu.sync_copy(agg_sh.at[pl.ds(s * RPT, RPT)],
                        out1.at[pl.ds(s * RPT, RPT)])


_sc_scatter = functools.partial(
    pl.kernel,
    out_type=[
        jax.ShapeDtypeStruct((N_NODES, DE), jnp.float32),
        jax.ShapeDtypeStruct((N_NODES, DE), jnp.float32),
    ],
    mesh=plsc.VectorSubcoreMesh(core_axis_name="c", subcore_axis_name="s"),
    compiler_params=pltpu.CompilerParams(use_tc_tiling_on_sc=False),
    scratch_types=[
        pltpu.VMEM((CH,), jnp.int32),
        pltpu.VMEM((CH,), jnp.int32),
        pltpu.VMEM((CH, DE), jnp.float32),
        pltpu.VMEM_SHARED((N_NODES, DE), jnp.float32),
        pltpu.SemaphoreType.DMA,
    ],
)(_sc_scatter_body)


def kernel(x, edge_index, W, B):
    src = edge_index[0]
    dst = edge_index[1]

    yext = pl.pallas_call(
        _mm_ext_body,
        out_shape=jax.ShapeDtypeStruct((N_NODES, DE), jnp.float32),
    )(x, W)

    zeros = jnp.zeros((N_NODES, DE), jnp.float32)
    a0, a1 = _sc_scatter(yext, zeros, src, dst)

    out = pl.pallas_call(
        _finish_body,
        out_shape=jax.ShapeDtypeStruct((N_NODES, D), jnp.float32),
    )(a0, a1, x, B)
    return out


# speedup vs baseline: 10.0885x; 1.9557x over previous
import functools

import jax
import jax.numpy as jnp
from jax import lax
from jax.experimental import pallas as pl
from jax.experimental.pallas import tpu as pltpu
from jax.experimental.pallas import tpu_sc as plsc

N_NODES = 10000
N_EDGES = 320000
D = 128
DE = 144

NC = 2
NS = 16
NW = NC * NS
EPW = N_EDGES // NW
CH = 125
NCH = EPW // CH
SCN = 8
NSC = NCH // SCN
RPT = N_NODES // NS


def _mm_ext_body(x_ref, w_ref, o_ref):
    y = lax.dot_general(x_ref[...], w_ref[...], (((1,), (1,)), ((), ())),
                        preferred_element_type=jnp.float32)
    o_ref[:, :D] = y
    col = lax.broadcasted_iota(jnp.int32, (x_ref.shape[0], DE - D), 1)
    o_ref[:, D:] = jnp.where(col == 0, 1.0, 0.0).astype(jnp.float32)


def _finish_body(a0_ref, a1_ref, x_ref, b_ref, o_ref):
    a = a0_ref[...] + a1_ref[...]
    agg = a[:, :D]
    deg = jnp.maximum(a[:, D:D + 1], 1.0)
    xb = lax.dot_general(x_ref[...], b_ref[...], (((1,), (1,)), ((), ())),
                         preferred_element_type=jnp.float32)
    o_ref[...] = agg / deg + xb


def _sc_scatter_body(yext, zeros_hbm, src_hbm, dst_hbm, out0, out1,
                     src_sl, dst_sl, r0, r1, agg_sh, g0, g1, isem):
    c = lax.axis_index("c")
    s = lax.axis_index("s")
    wid = s * NC + c
    rows = [r0, r1]
    gsem = [g0, g1]

    def idx_wait(slot):
        pltpu.make_async_copy(src_hbm.at[wid, pl.ds(0, SCN)],
                              src_sl.at[slot], isem).wait()
        pltpu.make_async_copy(dst_hbm.at[wid, pl.ds(0, SCN)],
                              dst_sl.at[slot], isem).wait()

    def gather(slot, k, b):
        pltpu.async_copy(yext.at[src_sl.at[slot, k]], rows[b], gsem[b])

    def gather_wait(b):
        pltpu.make_async_copy(yext.at[src_sl.at[0, 0]], rows[b],
                              gsem[b]).wait()

    def idx_load_async(sc, slot):
        pltpu.async_copy(src_hbm.at[wid, pl.ds(sc * SCN, SCN)],
                         src_sl.at[slot], isem)
        pltpu.async_copy(dst_hbm.at[wid, pl.ds(sc * SCN, SCN)],
                         dst_sl.at[slot], isem)

    pltpu.sync_copy(zeros_hbm.at[pl.ds(s * RPT, RPT)],
                    agg_sh.at[pl.ds(s * RPT, RPT)])

    pltpu.sync_copy(src_hbm.at[wid, pl.ds(0, SCN)], src_sl.at[0])
    pltpu.sync_copy(dst_hbm.at[wid, pl.ds(0, SCN)], dst_sl.at[0])
    idx_load_async(1, 1)
    plsc.subcore_barrier()

    gather(0, 0, 0)
    gather(0, 1, 1)

    def pair(r, carry):
        not_last = r < NSC // 2 - 1

        for k in range(SCN):
            b = k % 2
            gather_wait(b)
            pltpu.sync_copy(rows[b], agg_sh.at[dst_sl.at[0, k]], add=True)
            if k < SCN - 2:
                gather(0, k + 2, b)
            else:
                if k == SCN - 2:
                    idx_wait(1)
                gather(1, k + 2 - SCN, b)

        @pl.when(not_last)
        def _():
            idx_load_async(2 * r + 2, 0)

        for k in range(SCN):
            b = k % 2
            gather_wait(b)
            pltpu.sync_copy(rows[b], agg_sh.at[dst_sl.at[1, k]], add=True)
            if k < SCN - 2:
                gather(1, k + 2, b)
            else:
                if k == SCN - 2:
                    @pl.when(not_last)
                    def _():
                        idx_wait(0)

                @pl.when(not_last)
                def _():
                    gather(0, k + 2 - SCN, b)

        @pl.when(not_last)
        def _():
            idx_load_async(2 * r + 3, 1)
        return carry

    lax.fori_loop(0, NSC // 2, pair, 0)
    plsc.subcore_barrier()

    @pl.when(c == 0)
    def _():
        pltpu.sync_copy(agg_sh.at[pl.ds(s * RPT, RPT)],
                        out0.at[pl.ds(s * RPT, RPT)])

    @pl.when(c == 1)
    def _():
        pltpu.sync_copy(agg_sh.at[pl.ds(s * RPT, RPT)],
                        out1.at[pl.ds(s * RPT, RPT)])


_sc_scatter = functools.partial(
    pl.kernel,
    out_type=[
        jax.ShapeDtypeStruct((N_NODES, DE), jnp.float32),
        jax.ShapeDtypeStruct((N_NODES, DE), jnp.float32),
    ],
    mesh=plsc.VectorSubcoreMesh(core_axis_name="c", subcore_axis_name="s"),
    compiler_params=pltpu.CompilerParams(use_tc_tiling_on_sc=False),
    scratch_types=[
        pltpu.VMEM((2, SCN, CH), jnp.int32),
        pltpu.VMEM((2, SCN, CH), jnp.int32),
        pltpu.VMEM((CH, DE), jnp.float32),
        pltpu.VMEM((CH, DE), jnp.float32),
        pltpu.VMEM_SHARED((N_NODES, DE), jnp.float32),
        pltpu.SemaphoreType.DMA,
        pltpu.SemaphoreType.DMA,
        pltpu.SemaphoreType.DMA,
    ],
)(_sc_scatter_body)


def kernel(x, edge_index, W, B):
    src = edge_index[0].reshape(NW, NCH, CH)
    dst = edge_index[1].reshape(NW, NCH, CH)

    yext = pl.pallas_call(
        _mm_ext_body,
        out_shape=jax.ShapeDtypeStruct((N_NODES, DE), jnp.float32),
    )(x, W)

    zeros = jnp.zeros((N_NODES, DE), jnp.float32)
    a0, a1 = _sc_scatter(yext, zeros, src, dst)

    out = pl.pallas_call(
        _finish_body,
        out_shape=jax.ShapeDtypeStruct((N_NODES, D), jnp.float32),
    )(a0, a1, x, B)
    return out


# speedup vs baseline: 11.0409x; 1.0944x over previous
import functools

import jax
import jax.numpy as jnp
from jax import lax
from jax.experimental import pallas as pl
from jax.experimental.pallas import tpu as pltpu
from jax.experimental.pallas import tpu_sc as plsc

N_NODES = 10000
N_EDGES = 320000
D = 128
DE = 144

NC = 2
NS = 16
NW = NC * NS
EPW = N_EDGES // NW
CH = 125
NCH = EPW // CH
SCN = 8
NSC = NCH // SCN
RPT = N_NODES // NS


def _mm_ext_body(x_ref, w_ref, o_ref):
    y = lax.dot_general(x_ref[...], w_ref[...], (((1,), (1,)), ((), ())),
                        preferred_element_type=jnp.float32)
    o_ref[:, :D] = y
    col = lax.broadcasted_iota(jnp.int32, (x_ref.shape[0], DE - D), 1)
    o_ref[:, D:] = jnp.where(col == 0, 1.0, 0.0).astype(jnp.float32)


def _finish_body(a0_ref, a1_ref, x_ref, b_ref, o_ref):
    a = a0_ref[...] + a1_ref[...]
    agg = a[:, :D]
    deg = jnp.maximum(a[:, D:D + 1], 1.0)
    xb = lax.dot_general(x_ref[...], b_ref[...], (((1,), (1,)), ((), ())),
                         preferred_element_type=jnp.float32)
    o_ref[...] = agg / deg + xb


def _sc_scatter_body(yext, edge_hbm, out0, out1,
                     src_sl, dst_sl, r0, r1, agg_sh, g0, g1, isem):
    c = lax.axis_index("c")
    s = lax.axis_index("s")
    wid = s * NC + c
    rows = [r0, r1]
    gsem = [g0, g1]

    def idx_wait(slot):
        pltpu.make_async_copy(edge_hbm.at[0, wid, pl.ds(0, SCN)],
                              src_sl.at[slot], isem).wait()
        pltpu.make_async_copy(edge_hbm.at[1, wid, pl.ds(0, SCN)],
                              dst_sl.at[slot], isem).wait()

    def gather(slot, k, b):
        pltpu.async_copy(yext.at[src_sl.at[slot, k]], rows[b], gsem[b])

    def gather_wait(b):
        pltpu.make_async_copy(yext.at[src_sl.at[0, 0]], rows[b],
                              gsem[b]).wait()

    def idx_load_async(sc, slot):
        pltpu.async_copy(edge_hbm.at[0, wid, pl.ds(sc * SCN, SCN)],
                         src_sl.at[slot], isem)
        pltpu.async_copy(edge_hbm.at[1, wid, pl.ds(sc * SCN, SCN)],
                         dst_sl.at[slot], isem)

    zv = jnp.zeros((16,), jnp.float32)

    def zrow(i, carry):
        for j in range(DE // 16):
            r0[i, pl.ds(j * 16, 16)] = zv
        return carry

    lax.fori_loop(0, CH, zrow, 0)
    for p in range(RPT // CH):
        pltpu.sync_copy(r0, agg_sh.at[pl.ds(s * RPT + p * CH, CH)])

    pltpu.sync_copy(edge_hbm.at[0, wid, pl.ds(0, SCN)], src_sl.at[0])
    pltpu.sync_copy(edge_hbm.at[1, wid, pl.ds(0, SCN)], dst_sl.at[0])
    idx_load_async(1, 1)
    plsc.subcore_barrier()

    gather(0, 0, 0)
    gather(0, 1, 1)

    def pair(r, carry):
        not_last = r < NSC // 2 - 1

        for k in range(SCN):
            b = k % 2
            gather_wait(b)
            pltpu.sync_copy(rows[b], agg_sh.at[dst_sl.at[0, k]], add=True)
            if k < SCN - 2:
                gather(0, k + 2, b)
            else:
                if k == SCN - 2:
                    idx_wait(1)
                gather(1, k + 2 - SCN, b)

        @pl.when(not_last)
        def _():
            idx_load_async(2 * r + 2, 0)

        for k in range(SCN):
            b = k % 2
            gather_wait(b)
            pltpu.sync_copy(rows[b], agg_sh.at[dst_sl.at[1, k]], add=True)
            if k < SCN - 2:
                gather(1, k + 2, b)
            else:
                if k == SCN - 2:
                    @pl.when(not_last)
                    def _():
                        idx_wait(0)

                @pl.when(not_last)
                def _():
                    gather(0, k + 2 - SCN, b)

        @pl.when(not_last)
        def _():
            idx_load_async(2 * r + 3, 1)
        return carry

    lax.fori_loop(0, NSC // 2, pair, 0)
    plsc.subcore_barrier()

    @pl.when(c == 0)
    def _():
        pltpu.sync_copy(agg_sh.at[pl.ds(s * RPT, RPT)],
                        out0.at[pl.ds(s * RPT, RPT)])

    @pl.when(c == 1)
    def _():
        pltpu.sync_copy(agg_sh.at[pl.ds(s * RPT, RPT)],
                        out1.at[pl.ds(s * RPT, RPT)])


_sc_scatter = functools.partial(
    pl.kernel,
    out_type=[
        jax.ShapeDtypeStruct((N_NODES, DE), jnp.float32),
        jax.ShapeDtypeStruct((N_NODES, DE), jnp.float32),
    ],
    mesh=plsc.VectorSubcoreMesh(core_axis_name="c", subcore_axis_name="s"),
    compiler_params=pltpu.CompilerParams(use_tc_tiling_on_sc=False),
    scratch_types=[
        pltpu.VMEM((2, SCN, CH), jnp.int32),
        pltpu.VMEM((2, SCN, CH), jnp.int32),
        pltpu.VMEM((CH, DE), jnp.float32),
        pltpu.VMEM((CH, DE), jnp.float32),
        pltpu.VMEM_SHARED((N_NODES, DE), jnp.float32),
        pltpu.SemaphoreType.DMA,
        pltpu.SemaphoreType.DMA,
        pltpu.SemaphoreType.DMA,
    ],
)(_sc_scatter_body)


def kernel(x, edge_index, W, B):
    edges = edge_index.reshape(2, NW, NCH, CH)

    yext = pl.pallas_call(
        _mm_ext_body,
        out_shape=jax.ShapeDtypeStruct((N_NODES, DE), jnp.float32),
    )(x, W)

    a0, a1 = _sc_scatter(yext, edges)

    out = pl.pallas_call(
        _finish_body,
        out_shape=jax.ShapeDtypeStruct((N_NODES, D), jnp.float32),
    )(a0, a1, x, B)
    return out


# speedup vs baseline: 14.4117x; 1.3053x over previous
import functools

import jax
import jax.numpy as jnp
from jax import lax
from jax.experimental import pallas as pl
from jax.experimental.pallas import tpu as pltpu
from jax.experimental.pallas import tpu_sc as plsc

N_NODES = 10000
N_EDGES = 320000
D = 128

NC = 2
NS = 16
NW = NC * NS
EPW = N_EDGES // NW
CH = 125
NCH = EPW // CH
SCN = 8
NSC = NCH // SCN
RPT = N_NODES // NS
NDEG = 10240
DPT = NDEG // NS


def _finish_body(a0_ref, a1_ref, recip_ref, x_ref, w_ref, b_ref, o_ref):
    a = (a0_ref[...] + a1_ref[...]) * recip_ref[...]
    aw = lax.dot_general(a, w_ref[...], (((1,), (1,)), ((), ())),
                         preferred_element_type=jnp.float32)
    xb = lax.dot_general(x_ref[...], b_ref[...], (((1,), (1,)), ((), ())),
                         preferred_element_type=jnp.float32)
    o_ref[...] = aw + xb


def _sc_scatter_body(x_hbm, edge_hbm, out0, out1, outd0, outd1,
                     src_sl, dst_sl, r0, r1, ones_v, zdeg, agg_sh, deg_sh,
                     g0, g1, isem, dsem):
    c = lax.axis_index("c")
    s = lax.axis_index("s")
    wid = s * NC + c
    rows = [r0, r1]
    gsem = [g0, g1]

    def idx_wait(slot):
        pltpu.make_async_copy(edge_hbm.at[0, wid, pl.ds(0, SCN)],
                              src_sl.at[slot], isem).wait()
        pltpu.make_async_copy(edge_hbm.at[1, wid, pl.ds(0, SCN)],
                              dst_sl.at[slot], isem).wait()

    def gather(slot, k, b):
        pltpu.async_copy(x_hbm.at[src_sl.at[slot, k]], rows[b], gsem[b])

    def gather_wait(b):
        pltpu.make_async_copy(x_hbm.at[src_sl.at[0, 0]], rows[b],
                              gsem[b]).wait()

    def idx_load_async(sc, slot):
        pltpu.async_copy(edge_hbm.at[0, wid, pl.ds(sc * SCN, SCN)],
                         src_sl.at[slot], isem)
        pltpu.async_copy(edge_hbm.at[1, wid, pl.ds(sc * SCN, SCN)],
                         dst_sl.at[slot], isem)

    def scatter_chunk(slot, k, b):
        pltpu.sync_copy(rows[b], agg_sh.at[dst_sl.at[slot, k]], add=True)
        pltpu.async_copy(ones_v.at[pl.ds(0, CH)],
                         deg_sh.at[dst_sl.at[slot, k]], dsem, add=True)

    def deg_drain():
        for _ in range(SCN):
            pltpu.make_async_copy(ones_v.at[pl.ds(0, CH)],
                                  deg_sh.at[dst_sl.at[0, 0]], dsem).wait()

    fone = jnp.ones((16,), jnp.float32)
    fzero = jnp.zeros((16,), jnp.float32)

    def zrow(i, carry):
        for j in range(D // 16):
            r0[i, pl.ds(j * 16, 16)] = fzero
        return carry

    lax.fori_loop(0, CH, zrow, 0)
    for j in range(128 // 16):
        ones_v[pl.ds(j * 16, 16)] = fone
    for j in range(DPT // 16):
        zdeg[pl.ds(j * 16, 16)] = fzero

    for p in range(RPT // CH):
        pltpu.sync_copy(r0, agg_sh.at[pl.ds(s * RPT + p * CH, CH)])
    pltpu.sync_copy(zdeg, deg_sh.at[pl.ds(s * DPT, DPT)])

    pltpu.sync_copy(edge_hbm.at[0, wid, pl.ds(0, SCN)], src_sl.at[0])
    pltpu.sync_copy(edge_hbm.at[1, wid, pl.ds(0, SCN)], dst_sl.at[0])
    idx_load_async(1, 1)
    plsc.subcore_barrier()

    gather(0, 0, 0)
    gather(0, 1, 1)

    def pair(r, carry):
        not_last = r < NSC // 2 - 1

        for k in range(SCN):
            b = k % 2
            gather_wait(b)
            scatter_chunk(0, k, b)
            if k < SCN - 2:
                gather(0, k + 2, b)
            else:
                if k == SCN - 2:
                    idx_wait(1)
                gather(1, k + 2 - SCN, b)
        deg_drain()

        @pl.when(not_last)
        def _():
            idx_load_async(2 * r + 2, 0)

        for k in range(SCN):
            b = k % 2
            gather_wait(b)
            scatter_chunk(1, k, b)
            if k < SCN - 2:
                gather(1, k + 2, b)
            else:
                if k == SCN - 2:
                    @pl.when(not_last)
                    def _():
                        idx_wait(0)

                @pl.when(not_last)
                def _():
                    gather(0, k + 2 - SCN, b)
        deg_drain()

        @pl.when(not_last)
        def _():
            idx_load_async(2 * r + 3, 1)
        return carry

    lax.fori_loop(0, NSC // 2, pair, 0)
    plsc.subcore_barrier()

    @pl.when(c == 0)
    def _():
        pltpu.sync_copy(agg_sh.at[pl.ds(s * RPT, RPT)],
                        out0.at[pl.ds(s * RPT, RPT)])
        pltpu.sync_copy(deg_sh.at[pl.ds(s * DPT, DPT)],
                        outd0.at[pl.ds(s * DPT, DPT)])

    @pl.when(c == 1)
    def _():
        pltpu.sync_copy(agg_sh.at[pl.ds(s * RPT, RPT)],
                        out1.at[pl.ds(s * RPT, RPT)])
        pltpu.sync_copy(deg_sh.at[pl.ds(s * DPT, DPT)],
                        outd1.at[pl.ds(s * DPT, DPT)])


_sc_scatter = functools.partial(
    pl.kernel,
    out_type=[
        jax.ShapeDtypeStruct((N_NODES, D), jnp.float32),
        jax.ShapeDtypeStruct((N_NODES, D), jnp.float32),
        jax.ShapeDtypeStruct((NDEG,), jnp.float32),
        jax.ShapeDtypeStruct((NDEG,), jnp.float32),
    ],
    mesh=plsc.VectorSubcoreMesh(core_axis_name="c", subcore_axis_name="s"),
    compiler_params=pltpu.CompilerParams(use_tc_tiling_on_sc=False),
    scratch_types=[
        pltpu.VMEM((2, SCN, CH), jnp.int32),
        pltpu.VMEM((2, SCN, CH), jnp.int32),
        pltpu.VMEM((CH, D), jnp.float32),
        pltpu.VMEM((CH, D), jnp.float32),
        pltpu.VMEM((128,), jnp.float32),
        pltpu.VMEM((DPT,), jnp.float32),
        pltpu.VMEM_SHARED((N_NODES, D), jnp.float32),
        pltpu.VMEM_SHARED((NDEG,), jnp.float32),
        pltpu.SemaphoreType.DMA,
        pltpu.SemaphoreType.DMA,
        pltpu.SemaphoreType.DMA,
        pltpu.SemaphoreType.DMA,
    ],
)(_sc_scatter_body)


def kernel(x, edge_index, W, B):
    edges = edge_index.reshape(2, NW, NCH, CH)

    a0, a1, d0, d1 = _sc_scatter(x, edges)

    recip = (1.0 / jnp.maximum(d0[:N_NODES] + d1[:N_NODES], 1.0))[:, None]

    out = pl.pallas_call(
        _finish_body,
        out_shape=jax.ShapeDtypeStruct((N_NODES, D), jnp.float32),
    )(a0, a1, recip, x, W, B)
    return out
